# baseline (speedup 1.0000x reference)
import functools

import jax
import jax.numpy as jnp
from jax import lax
from jax.experimental import pallas as pl
from jax.experimental.pallas import tpu as pltpu
from jax.experimental.pallas import tpu_sc as plsc

_B = 1024
_V = 100000

_info = plsc.get_sparse_core_info()
_L = _info.num_lanes
_NW = 16
_BPW = _B // _NW

_LN2 = 0.6931471805599453

_mesh = plsc.VectorSubcoreMesh(
    core_axis_name="c", subcore_axis_name="s", num_cores=1
)


def _neg_log(v):
    bits = lax.bitcast_convert_type(v, jnp.int32)
    e = lax.convert_element_type(
        lax.shift_right_logical(bits, 23) - 127, jnp.float32
    )
    m = lax.bitcast_convert_type(
        (bits & 0x7FFFFF) | 0x3F800000, jnp.float32
    )
    z = (m - 1.0) / (m + 1.0)
    z2 = z * z
    p = 1.0 + z2 * (
        1.0 / 3.0 + z2 * (1.0 / 5.0 + z2 * (1.0 / 7.0 + z2 * (1.0 / 9.0)))
    )
    return -(e * _LN2 + 2.0 * z * p)


@functools.partial(
    pl.kernel,
    mesh=_mesh,
    out_type=jax.ShapeDtypeStruct((1,), jnp.float32),
    scratch_types=[
        pltpu.VMEM((_BPW,), jnp.int32),
        pltpu.VMEM((_BPW, _B), jnp.float32),
        pltpu.VMEM((_L,), jnp.float32),
        pltpu.VMEM((_L,), jnp.float32),
        pltpu.VMEM_SHARED((_L,), jnp.float32),
        pltpu.SemaphoreType.DMA,
    ],
    compiler_params=pltpu.CompilerParams(
        needs_layout_passes=False, skip_device_barrier=True,
        disable_semaphore_checks=True
    ),
)
def _sc_loss(prdt_hbm, trg_hbm, out_hbm, idx_v, rows_v, acc_v, red_v,
             shared, sem):
    wid = lax.axis_index("s")
    base = wid * _BPW

    @pl.when(wid == 0)
    def _zero():
        acc_v[...] = jnp.zeros((_L,), jnp.float32)
        pltpu.sync_copy(acc_v, shared)

    plsc.subcore_barrier()
    pltpu.sync_copy(trg_hbm.at[pl.ds(base, _BPW)], idx_v)
    pltpu.async_copy(prdt_hbm.at[idx_v], rows_v, sem).wait()
    acc = jnp.zeros((_L,), jnp.float32)
    for c in range(_BPW // _L):
        rows = c * _L + lax.broadcasted_iota(jnp.int32, (_L,), 0)
        cols = base + rows
        acc = acc + _neg_log(plsc.load_gather(rows_v, [rows, cols]))
    acc_v[...] = acc
    lanes = lax.broadcasted_iota(jnp.int32, (_L,), 0)
    pltpu.sync_copy(acc_v, shared.at[lanes], add=True)
    plsc.subcore_barrier()

    @pl.when(wid == 0)
    def _reduce():
        pltpu.sync_copy(shared, red_v)
        total = jnp.sum(red_v[...])
        red_v[...] = jnp.full((_L,), total, jnp.float32)
        pltpu.sync_copy(red_v.at[pl.ds(0, 1)], out_hbm)


def kernel(prd, trg):
    vals = _sc_loss(prd.T, trg.astype(jnp.int32))
    return vals.reshape(())


# speedup vs baseline: 1.0997x; 1.0997x over previous
import functools

import jax
import jax.numpy as jnp
from jax import lax
from jax.experimental import pallas as pl
from jax.experimental.pallas import tpu as pltpu
from jax.experimental.pallas import tpu_sc as plsc

_B = 1024
_V = 100000

_info = plsc.get_sparse_core_info()
_L = _info.num_lanes
_NW = 16
_BPW = _B // _NW

_LN2 = 0.6931471805599453

_mesh = plsc.VectorSubcoreMesh(
    core_axis_name="c", subcore_axis_name="s", num_cores=1
)


def _neg_log(v):
    bits = lax.bitcast_convert_type(v, jnp.int32)
    e = lax.convert_element_type(
        lax.shift_right_logical(bits, 23) - 127, jnp.float32
    )
    m = lax.bitcast_convert_type(
        (bits & 0x7FFFFF) | 0x3F800000, jnp.float32
    )
    z = (m - 1.0) / (m + 1.0)
    z2 = z * z
    p = 1.0 + z2 * (
        1.0 / 3.0 + z2 * (1.0 / 5.0 + z2 * (1.0 / 7.0 + z2 * (1.0 / 9.0)))
    )
    return -(e * _LN2 + 2.0 * z * p)


@functools.partial(
    pl.kernel,
    mesh=_mesh,
    out_type=jax.ShapeDtypeStruct((1,), jnp.float32),
    scratch_types=[
        pltpu.VMEM((_BPW,), jnp.int32),
        pltpu.VMEM((_BPW, 128), jnp.float32),
        pltpu.VMEM((_L,), jnp.float32),
        pltpu.VMEM((_L,), jnp.float32),
        pltpu.VMEM_SHARED((_L,), jnp.float32),
        pltpu.SemaphoreType.DMA,
    ],
    compiler_params=pltpu.CompilerParams(
        needs_layout_passes=False, skip_device_barrier=True,
        disable_semaphore_checks=True
    ),
)
def _sc_loss(prdt_hbm, trg_hbm, out_hbm, idx_v, rows_v, acc_v, red_v,
             shared, sem):
    wid = lax.axis_index("s")
    base = wid * _BPW

    @pl.when(wid == 0)
    def _zero():
        acc_v[...] = jnp.zeros((_L,), jnp.float32)
        pltpu.sync_copy(acc_v, shared)

    plsc.subcore_barrier()
    pltpu.sync_copy(trg_hbm.at[pl.ds(base, _BPW)], idx_v)
    colb = pl.multiple_of((base // 128) * 128, 128)
    pltpu.async_copy(prdt_hbm.at[idx_v, pl.ds(colb, 128)], rows_v, sem).wait()
    acc = jnp.zeros((_L,), jnp.float32)
    for c in range(_BPW // _L):
        rows = c * _L + lax.broadcasted_iota(jnp.int32, (_L,), 0)
        cols = (base + rows) & 127
        acc = acc + _neg_log(plsc.load_gather(rows_v, [rows, cols]))
    acc_v[...] = acc
    lanes = lax.broadcasted_iota(jnp.int32, (_L,), 0)
    pltpu.sync_copy(acc_v, shared.at[lanes], add=True)
    plsc.subcore_barrier()

    @pl.when(wid == 0)
    def _reduce():
        pltpu.sync_copy(shared, red_v)
        total = jnp.sum(red_v[...])
        red_v[...] = jnp.full((_L,), total, jnp.float32)
        pltpu.sync_copy(red_v.at[pl.ds(0, 1)], out_hbm)


def kernel(prd, trg):
    vals = _sc_loss(prd.T, trg.astype(jnp.int32))
    return vals.reshape(())
